# Initial kernel scaffold; baseline (speedup 1.0000x reference)
import jax
import jax.numpy as jnp
from jax.experimental import pallas as pl


def kernel(x_a, edge_index_a2a, edge_attr_a2a, W_g, b_g, W_f, b_f, W_c, b_c):
    raise NotImplementedError("write your pallas kernel here")


# speedup vs baseline: 10.4784x; 10.4784x over previous
import functools

import jax
import jax.numpy as jnp
from jax import lax
from jax.experimental import pallas as pl
from jax.experimental.pallas import tpu as pltpu
from jax.experimental.pallas import tpu_sc as plsc

N = 10000
E = 320000
C = 128
NT = 32
EPT = E // NT
_PREC = lax.Precision.HIGHEST


def _prep_body(x_ref, attr_ref, wgi, wgj, wge, wfx, wfa, wc, bg, bf, bc,
               p_out, base_out, beta_out, q_out):
    wc_v = wc[...]
    v2 = jnp.dot(wfa[...], wc_v, precision=_PREC)
    u2 = jnp.dot(wfx[...], wc_v, precision=_PREC)
    a2 = jnp.dot(wgi[...], v2, precision=_PREC)
    w2 = jnp.dot(wgj[...], v2, precision=_PREC)
    t2 = jnp.dot(wge[...], v2, precision=_PREC)
    g0 = jnp.dot(bg[...], v2, precision=_PREC)
    c0 = jnp.dot(bf[...], wc_v, precision=_PREC) + bc[...]
    x = x_ref[...]
    p_out[...] = jnp.dot(x, w2, precision=_PREC)
    base_out[...] = jnp.dot(x, u2, precision=_PREC) + c0
    beta_out[...] = jnp.dot(x, a2, precision=_PREC) + g0
    r = lax.broadcasted_iota(jnp.int32, (C, 32), 0)
    cidx = lax.broadcasted_iota(jnp.int32, (C, 32), 1)
    blk = ((r // 4) == cidx).astype(jnp.float32)
    r4 = lax.broadcasted_iota(jnp.int32, (C, 4), 0)
    j4 = lax.broadcasted_iota(jnp.int32, (C, 4), 1)
    sel = ((r4 % 4) == j4).astype(jnp.float32)
    tcol = jnp.dot(sel, t2, precision=_PREC)
    q_out[...] = jnp.dot(attr_ref[...], blk * tcol, precision=_PREC)


_prep_call = pl.pallas_call(
    _prep_body,
    out_shape=(
        jax.ShapeDtypeStruct((N, 1), jnp.float32),
        jax.ShapeDtypeStruct((N, 1), jnp.float32),
        jax.ShapeDtypeStruct((N, 1), jnp.float32),
        jax.ShapeDtypeStruct((E * 4 // C, 32), jnp.float32),
    ),
    compiler_params=pltpu.CompilerParams(vmem_limit_bytes=100 * 1024 * 1024),
)


N_PAD = 10240
ROWS = 80
EPT_PAD = ROWS * 128
E_PAD = NT * EPT_PAD
N_SLICE = N_PAD // 16


def _edge_body(p_hbm, src_hbm, dst_hbm, q_hbm, acc_out, cnt_out,
               p_v, src_v, dst_v, q_v, vals_v, z_v, ones_v, acc_sh, cnt_sh):
    cid = lax.axis_index("c")
    sid = lax.axis_index("s")
    wid = sid * 2 + cid
    row0 = wid * ROWS
    pltpu.sync_copy(p_hbm, p_v)
    pltpu.sync_copy(src_hbm.at[pl.ds(row0, ROWS)], src_v)
    pltpu.sync_copy(dst_hbm.at[pl.ds(row0, ROWS)], dst_v)
    pltpu.sync_copy(q_hbm.at[pl.ds(row0, ROWS)], q_v)

    zeros16 = jnp.zeros((16,), jnp.float32)
    ones16 = jnp.ones((16,), jnp.float32)

    def zbody(i, carry):
        z_v[pl.ds(i * 16, 16)] = zeros16
        return carry

    lax.fori_loop(0, N_SLICE // 16, zbody, 0)
    for j in range(8):
        ones_v[pl.ds(j * 16, 16)] = ones16
    sl = pl.ds(sid * N_SLICE, N_SLICE)
    pltpu.sync_copy(z_v, acc_sh.at[sl])
    pltpu.sync_copy(z_v, cnt_sh.at[sl])
    plsc.subcore_barrier()

    def vbody(c, carry):
        for j in range(8):
            ls = pl.ds(j * 16, 16)
            s16 = src_v[c, ls]
            q16 = q_v[c, ls]
            pv = plsc.load_gather(p_v, [s16])
            vals_v[c, ls] = pv + q16
        return carry

    lax.fori_loop(0, ROWS, vbody, 0)

    def sbody(c, carry):
        idx = dst_v.at[c]
        pltpu.sync_copy(vals_v.at[c], acc_sh.at[idx], add=True)
        pltpu.sync_copy(ones_v, cnt_sh.at[idx], add=True)
        return carry

    lax.fori_loop(0, ROWS, sbody, 0)
    plsc.subcore_barrier()

    pltpu.sync_copy(acc_sh.at[sl], acc_out.at[cid, sl])
    pltpu.sync_copy(cnt_sh.at[sl], cnt_out.at[cid, sl])


def _make_edge_call():
    mesh = plsc.VectorSubcoreMesh(core_axis_name="c", subcore_axis_name="s")
    return pl.kernel(
        _edge_body,
        out_type=(
            jax.ShapeDtypeStruct((2, N_PAD), jnp.float32),
            jax.ShapeDtypeStruct((2, N_PAD), jnp.float32),
        ),
        mesh=mesh,
        compiler_params=pltpu.CompilerParams(needs_layout_passes=False),
        scratch_types=[
            pltpu.VMEM((N,), jnp.float32),
            pltpu.VMEM((ROWS, 128), jnp.int32),
            pltpu.VMEM((ROWS, 128), jnp.int32),
            pltpu.VMEM((ROWS, 128), jnp.float32),
            pltpu.VMEM((ROWS, 128), jnp.float32),
            pltpu.VMEM((N_SLICE,), jnp.float32),
            pltpu.VMEM((128,), jnp.float32),
            pltpu.VMEM_SHARED((N_PAD,), jnp.float32),
            pltpu.VMEM_SHARED((N_PAD,), jnp.float32),
        ],
    )


def _post_body(accp, cntp, base, beta, probs_out, y_out):
    acc = jnp.sum(accp[...], axis=0)
    cnt = jnp.sum(cntp[...], axis=0)
    m = (cnt > 0.0).astype(jnp.float32)
    logit = base[...] + m * beta[...] + acc / jnp.maximum(cnt, 1.0)
    probs = jax.nn.sigmoid(logit)
    probs_out[...] = probs
    y_out[...] = (probs > 0.5).astype(jnp.uint8)


_post_call = pl.pallas_call(
    _post_body,
    out_shape=(
        jax.ShapeDtypeStruct((N_PAD,), jnp.float32),
        jax.ShapeDtypeStruct((N_PAD,), jnp.uint8),
    ),
)


def kernel(x_a, edge_index_a2a, edge_attr_a2a, W_g, b_g, W_f, b_f, W_c, b_c):
    src = edge_index_a2a[0]
    dst = edge_index_a2a[1]
    attr_rs = edge_attr_a2a.reshape(E * 4 // C, C)
    p2, base2, beta2, q_rs = _prep_call(
        x_a, attr_rs,
        W_g[:C], W_g[C:2 * C], W_g[2 * C:],
        W_f[:C], W_f[C:],
        W_c, b_g.reshape(1, C), b_f.reshape(1, C), b_c.reshape(1, 1),
    )
    p = p2.reshape(N)
    pad = E_PAD - E
    src_p = jnp.concatenate([src, jnp.zeros((pad,), jnp.int32)]).reshape(NT * ROWS, 128)
    dst_p = jnp.concatenate(
        [dst, jnp.full((pad,), N_PAD - 1, jnp.int32)]).reshape(NT * ROWS, 128)
    q_p = jnp.concatenate(
        [q_rs.reshape(E), jnp.zeros((pad,), jnp.float32)]).reshape(NT * ROWS, 128)
    accp, cntp = _make_edge_call()(p, src_p, dst_p, q_p)
    npad = N_PAD - N
    base_p = jnp.concatenate([base2.reshape(N), jnp.zeros((npad,), jnp.float32)])
    beta_p = jnp.concatenate([beta2.reshape(N), jnp.zeros((npad,), jnp.float32)])
    probs, y8 = _post_call(accp, cntp, base_p, beta_p)
    return (y8[:N].astype(jnp.bool_).reshape(N, 1), probs[:N].reshape(N, 1))
